# baseline (speedup 1.0000x reference)
import functools

import jax
import jax.numpy as jnp
from jax.experimental import pallas as pl
from jax.experimental.pallas import tpu as pltpu

N_EMBD = 1024
N_HEAD = 16
HEAD_DIM = N_EMBD // N_HEAD
BLOCK = 64
SEQ = 2048
NB = SEQ // BLOCK
TQ = 512


def _mm_bias_kernel(x_ref, w_ref, b_ref, o_ref, *, out_dtype, out_scale):
    w = w_ref[...].astype(jnp.bfloat16)
    acc = jax.lax.dot_general(
        x_ref[...], w, (((1,), (1,)), ((), ())),
        preferred_element_type=jnp.float32,
    ) + b_ref[...]
    if out_scale is not None:
        acc = acc * out_scale
    o_ref[...] = acc.astype(out_dtype)


def _mm_bias(x, w, b, tm, tn, out_dtype, out_scale=None):
    m, k = x.shape
    n, _ = w.shape
    grid = (n // tn, m // tm)
    return pl.pallas_call(
        functools.partial(_mm_bias_kernel, out_dtype=out_dtype,
                          out_scale=out_scale),
        grid=grid,
        in_specs=[
            pl.BlockSpec((tm, k), lambda j, i: (i, 0)),
            pl.BlockSpec((tn, k), lambda j, i: (j, 0)),
            pl.BlockSpec((1, tn), lambda j, i: (0, j)),
        ],
        out_specs=pl.BlockSpec((tm, tn), lambda j, i: (i, j)),
        out_shape=jax.ShapeDtypeStruct((m, n), out_dtype),
        compiler_params=pltpu.CompilerParams(
            dimension_semantics=("parallel", "parallel")
        ),
    )(x, w, b.reshape(1, -1))


def _head_attn(q, k, v, b1, b2, t, kv_len):
    nbloc = kv_len // BLOCK
    s = jax.lax.dot_general(
        q, k, (((1,), (1,)), ((), ())), preferred_element_type=jnp.float32
    )
    e = jnp.exp(s).astype(jnp.bfloat16)
    denom = jnp.dot(e, b1, preferred_element_type=jnp.float32)

    row = jax.lax.broadcasted_iota(jnp.int32, (TQ, nbloc), 0)
    qblk = t * (TQ // BLOCK) + row // BLOCK
    col = jax.lax.broadcasted_iota(jnp.int32, (TQ, nbloc), 1)
    dinv = jnp.where(col <= qblk, 1.0 / denom, 0.0)

    denomb = jnp.dot(
        dinv.astype(jnp.bfloat16), b2, preferred_element_type=jnp.float32
    )
    p = e * denomb.astype(jnp.bfloat16)
    return jnp.dot(p, v, preferred_element_type=jnp.float32).astype(jnp.bfloat16)


def _attn_kernel(q_ref, k_ref, v_ref, b1_ref, b2_ref, o_ref, *, t_off, kv_len):
    t = t_off + pl.program_id(1)
    b1 = b1_ref[...]
    b2 = b2_ref[...]
    outs = []
    for i in (0, 1):
        sl = slice(HEAD_DIM * i, HEAD_DIM * (i + 1))
        outs.append(
            _head_attn(q_ref[:, sl], k_ref[:, sl], v_ref[:, sl], b1, b2, t,
                       kv_len)
        )
    o_ref[...] = jnp.concatenate(outs, axis=1)


def _attention(q, k, v, b1, b2, t_off, nt, kv_len):
    npair = N_HEAD // 2
    nbloc = kv_len // BLOCK
    grid = (npair, nt)
    return pl.pallas_call(
        functools.partial(_attn_kernel, t_off=t_off, kv_len=kv_len),
        grid=grid,
        in_specs=[
            pl.BlockSpec((TQ, 2 * HEAD_DIM), lambda p, t: (t + t_off, p)),
            pl.BlockSpec((kv_len, 2 * HEAD_DIM), lambda p, t: (0, p)),
            pl.BlockSpec((kv_len, 2 * HEAD_DIM), lambda p, t: (0, p)),
            pl.BlockSpec((kv_len, nbloc), lambda p, t: (0, 0)),
            pl.BlockSpec((nbloc, kv_len), lambda p, t: (0, 0)),
        ],
        out_specs=pl.BlockSpec((TQ, 2 * HEAD_DIM), lambda p, t: (t, p)),
        out_shape=jax.ShapeDtypeStruct((nt * TQ, N_EMBD), jnp.bfloat16),
        compiler_params=pltpu.CompilerParams(
            dimension_semantics=("parallel", "arbitrary")
        ),
    )(q, k, v, b1[:kv_len, :nbloc], b2[:nbloc, :kv_len])


def kernel(x, Wq, bq, Wk, bk, Wv, bv, Wo, bo):
    B, S, E = x.shape
    x2 = x.reshape(S, E).astype(jnp.bfloat16)

    scale = 1.0 / (HEAD_DIM ** 0.5)
    q = _mm_bias(x2, Wq, bq, tm=256, tn=512, out_dtype=jnp.bfloat16,
                 out_scale=scale)
    k = _mm_bias(x2, Wk, bk, tm=256, tn=512, out_dtype=jnp.bfloat16)
    v = _mm_bias(x2, Wv, bv, tm=256, tn=512, out_dtype=jnp.bfloat16)

    blk_ids = jnp.arange(SEQ, dtype=jnp.int32) // BLOCK
    b1 = (blk_ids[:, None] == jnp.arange(NB, dtype=jnp.int32)[None, :]).astype(
        jnp.bfloat16
    )
    b2 = b1.T

    nt = SEQ // TQ
    parts = [
        _attention(q, k, v, b1, b2, t, 1, (t + 1) * TQ) for t in range(nt)
    ]
    y = jnp.concatenate(parts, axis=0)

    out = _mm_bias(y, Wo, bo, tm=256, tn=512, out_dtype=jnp.float32)
    return out.reshape(B, S, E)
